# Initial kernel scaffold; baseline (speedup 1.0000x reference)
import jax
import jax.numpy as jnp
from jax.experimental import pallas as pl


def kernel(adj_matrix, block_assignments, block_probs):
    raise NotImplementedError("write your pallas kernel here")


# speedup vs baseline: 6.1179x; 6.1179x over previous
import functools

import jax
import jax.numpy as jnp
from jax.experimental import pallas as pl
from jax.experimental.pallas import tpu as pltpu

N = 4096
K = 32
EPS = 1e-10
BR = 512
PREC = jax.lax.Precision.HIGHEST


def _estep_body(z_ref, l1t_ref, l0t_ref, adj_ref, resp_ref, asg_ref,
                l1z_scr, l0z_scr):
    i = pl.program_id(0)

    @pl.when(i == 0)
    def _init():
        kio = jax.lax.broadcasted_iota(jnp.int32, (N, K), 1)
        onehot = (z_ref[...] == kio).astype(jnp.float32)
        l1z_scr[...] = jnp.dot(onehot, l1t_ref[...],
                               preferred_element_type=jnp.float32, precision=PREC)
        l0z_scr[...] = jnp.dot(onehot, l0t_ref[...],
                               preferred_element_type=jnp.float32, precision=PREC)

    a = adj_ref[...]
    logp = (jnp.dot(a, l1z_scr[...], preferred_element_type=jnp.float32)
            + jnp.dot(1.0 - a, l0z_scr[...], preferred_element_type=jnp.float32))
    logp = logp - l0z_scr[pl.ds(i * BR, BR), :]
    m = jnp.max(logp, axis=1, keepdims=True)
    e = jnp.exp(logp - m)
    r = e / jnp.sum(e, axis=1, keepdims=True)
    resp_ref[...] = r
    mx = jnp.max(r, axis=1, keepdims=True)
    kio = jax.lax.broadcasted_iota(jnp.int32, (BR, K), 1)
    asg_ref[...] = jnp.min(jnp.where(r == mx, kio, K), axis=1, keepdims=True)


def _mstep_body(z_ref, adj_ref, m_ref, nrow_ref, ncol_ref, e_scr):
    i = pl.program_id(0)

    @pl.when(i == 0)
    def _init():
        kio = jax.lax.broadcasted_iota(jnp.int32, (N, K), 1)
        e_full = (z_ref[...] == kio).astype(jnp.float32)
        e_scr[...] = e_full
        nrow_ref[...] = jnp.sum(e_full, axis=0, keepdims=True)
        ones = jnp.ones((N, 1), jnp.float32)
        ncol_ref[...] = jax.lax.dot_general(
            e_full, ones, (((0,), (0,)), ((), ())),
            preferred_element_type=jnp.float32)
        m_ref[...] = jnp.zeros((K, K), jnp.float32)

    r = jnp.dot(adj_ref[...], e_scr[...],
                preferred_element_type=jnp.float32)
    e_blk = e_scr[pl.ds(i * BR, BR), :]
    m_ref[...] += jax.lax.dot_general(
        e_blk, r, (((0,), (0,)), ((), ())),
        preferred_element_type=jnp.float32)


def _final_body(m_ref, nrow_ref, ncol_ref, p_ref, ll_ref):
    m = m_ref[...]
    nrow = nrow_ref[...]
    ncol = ncol_ref[...]
    ri = jax.lax.broadcasted_iota(jnp.int32, (K, K), 0)
    ci = jax.lax.broadcasted_iota(jnp.int32, (K, K), 1)
    eye = ri == ci
    counts_raw = ncol * nrow - jnp.where(eye, nrow, 0.0)
    counts = jnp.where(counts_raw == 0.0, 1.0, counts_raw)
    p = m / counts
    l1p = jnp.log(p + EPS)
    l0p = jnp.log(1.0 - p + EPS)
    ll = 0.5 * jnp.sum(m * l1p + (counts_raw - m) * l0p)
    p_ref[...] = p
    ll_ref[...] = jnp.full((1, 1), ll, jnp.float32)


def kernel(adj_matrix, block_assignments, block_probs):
    z2 = block_assignments.astype(jnp.int32).reshape(N, 1)
    l1t = jnp.log(block_probs + EPS).T
    l0t = jnp.log(1.0 - block_probs + EPS).T

    grid = (N // BR,)
    resp, asg = pl.pallas_call(
        _estep_body,
        grid=grid,
        in_specs=[
            pl.BlockSpec((N, 1), lambda i: (0, 0)),
            pl.BlockSpec((K, K), lambda i: (0, 0)),
            pl.BlockSpec((K, K), lambda i: (0, 0)),
            pl.BlockSpec((BR, N), lambda i: (i, 0)),
        ],
        out_specs=[
            pl.BlockSpec((BR, K), lambda i: (i, 0)),
            pl.BlockSpec((BR, 1), lambda i: (i, 0)),
        ],
        out_shape=[
            jax.ShapeDtypeStruct((N, K), jnp.float32),
            jax.ShapeDtypeStruct((N, 1), jnp.int32),
        ],
        scratch_shapes=[
            pltpu.VMEM((N, K), jnp.float32),
            pltpu.VMEM((N, K), jnp.float32),
        ],
    )(z2, l1t, l0t, adj_matrix)

    new_assign = asg.reshape(N)

    m_mat, nrow, ncol = pl.pallas_call(
        _mstep_body,
        grid=grid,
        in_specs=[
            pl.BlockSpec((N, 1), lambda i: (0, 0)),
            pl.BlockSpec((BR, N), lambda i: (i, 0)),
        ],
        out_specs=[
            pl.BlockSpec((K, K), lambda i: (0, 0)),
            pl.BlockSpec((1, K), lambda i: (0, 0)),
            pl.BlockSpec((K, 1), lambda i: (0, 0)),
        ],
        out_shape=[
            jax.ShapeDtypeStruct((K, K), jnp.float32),
            jax.ShapeDtypeStruct((1, K), jnp.float32),
            jax.ShapeDtypeStruct((K, 1), jnp.float32),
        ],
        scratch_shapes=[
            pltpu.VMEM((N, K), jnp.float32),
        ],
    )(asg, adj_matrix)

    p, ll = pl.pallas_call(
        _final_body,
        out_shape=[
            jax.ShapeDtypeStruct((K, K), jnp.float32),
            jax.ShapeDtypeStruct((1, 1), jnp.float32),
        ],
    )(m_mat, nrow, ncol)

    return resp, new_assign, p, ll.reshape(())


# speedup vs baseline: 7.1620x; 1.1707x over previous
import jax
import jax.numpy as jnp
from jax.experimental import pallas as pl
from jax.experimental.pallas import tpu as pltpu

N = 4096
K = 32
EPS = 1e-10
BR = 256
NB = N // BR
PREC = jax.lax.Precision.HIGHEST


def _em_body(z_ref, l1t_ref, l0t_ref, adj_ref,
             resp_ref, asg_ref, m_ref, nrow_ref, ncol_ref,
             l1z_scr, l0z_scr, abf_scr, zn_scr, e_scr, ebf_scr):
    i = pl.program_id(0)

    @pl.when(i == 0)
    def _init_e():
        kio = jax.lax.broadcasted_iota(jnp.int32, (N, K), 1)
        onehot = (z_ref[...] == kio).astype(jnp.float32)
        l1z_scr[...] = jnp.dot(onehot, l1t_ref[...],
                               preferred_element_type=jnp.float32, precision=PREC)
        l0z_scr[...] = jnp.dot(onehot, l0t_ref[...],
                               preferred_element_type=jnp.float32, precision=PREC)

    @pl.when(i < NB)
    def _estep():
        a = adj_ref[...]
        abf_scr[pl.ds(i * BR, BR), :] = a.astype(jnp.float8_e4m3fn)
        logp = (jnp.dot(a, l1z_scr[...], preferred_element_type=jnp.float32)
                + jnp.dot(1.0 - a, l0z_scr[...],
                          preferred_element_type=jnp.float32))
        logp = logp - l0z_scr[pl.ds(i * BR, BR), :]
        m = jnp.max(logp, axis=1, keepdims=True)
        e = jnp.exp(logp - m)
        r = e / jnp.sum(e, axis=1, keepdims=True)
        resp_ref[...] = r
        mx = jnp.max(r, axis=1, keepdims=True)
        kio = jax.lax.broadcasted_iota(jnp.int32, (BR, K), 1)
        asg = jnp.min(jnp.where(r == mx, kio, K), axis=1, keepdims=True)
        asg_ref[...] = asg
        zn_scr[pl.ds(i * BR, BR), :] = asg

    @pl.when(i == NB)
    def _init_m():
        kio = jax.lax.broadcasted_iota(jnp.int32, (N, K), 1)
        e_full = (zn_scr[...] == kio).astype(jnp.float32)
        e_scr[...] = e_full
        ebf_scr[...] = e_full.astype(jnp.float8_e4m3fn)
        nrow_ref[...] = jnp.sum(e_full, axis=0, keepdims=True)
        ones = jnp.ones((N, 1), jnp.float32)
        ncol_ref[...] = jax.lax.dot_general(
            e_full, ones, (((0,), (0,)), ((), ())),
            preferred_element_type=jnp.float32)
        m_ref[...] = jnp.zeros((K, K), jnp.float32)

    @pl.when(i >= NB)
    def _mstep():
        j = i - NB
        ab = abf_scr[pl.ds(j * BR, BR), :]
        r = jnp.dot(ab, ebf_scr[...], preferred_element_type=jnp.float32)
        e_blk = e_scr[pl.ds(j * BR, BR), :]
        m_ref[...] += jax.lax.dot_general(
            e_blk, r, (((0,), (0,)), ((), ())),
            preferred_element_type=jnp.float32)


def _final_body(m_ref, nrow_ref, ncol_ref, p_ref, ll_ref):
    m = m_ref[...]
    nrow = nrow_ref[...]
    ncol = ncol_ref[...]
    ri = jax.lax.broadcasted_iota(jnp.int32, (K, K), 0)
    ci = jax.lax.broadcasted_iota(jnp.int32, (K, K), 1)
    eye = ri == ci
    counts_raw = ncol * nrow - jnp.where(eye, nrow, 0.0)
    counts = jnp.where(counts_raw == 0.0, 1.0, counts_raw)
    p = m / counts
    l1p = jnp.log(p + EPS)
    l0p = jnp.log(1.0 - p + EPS)
    ll = 0.5 * jnp.sum(m * l1p + (counts_raw - m) * l0p)
    p_ref[...] = p
    ll_ref[...] = jnp.full((1, 1), ll, jnp.float32)


def kernel(adj_matrix, block_assignments, block_probs):
    z2 = block_assignments.astype(jnp.int32).reshape(N, 1)
    l1t = jnp.log(block_probs + EPS).T
    l0t = jnp.log(1.0 - block_probs + EPS).T

    last = NB - 1
    resp, asg, m_mat, nrow, ncol = pl.pallas_call(
        _em_body,
        grid=(2 * NB,),
        in_specs=[
            pl.BlockSpec((N, 1), lambda i: (0, 0)),
            pl.BlockSpec((K, K), lambda i: (0, 0)),
            pl.BlockSpec((K, K), lambda i: (0, 0)),
            pl.BlockSpec((BR, N), lambda i: (jnp.minimum(i, last), 0)),
        ],
        out_specs=[
            pl.BlockSpec((BR, K), lambda i: (jnp.minimum(i, last), 0)),
            pl.BlockSpec((BR, 1), lambda i: (jnp.minimum(i, last), 0)),
            pl.BlockSpec((K, K), lambda i: (0, 0)),
            pl.BlockSpec((1, K), lambda i: (0, 0)),
            pl.BlockSpec((K, 1), lambda i: (0, 0)),
        ],
        out_shape=[
            jax.ShapeDtypeStruct((N, K), jnp.float32),
            jax.ShapeDtypeStruct((N, 1), jnp.int32),
            jax.ShapeDtypeStruct((K, K), jnp.float32),
            jax.ShapeDtypeStruct((1, K), jnp.float32),
            jax.ShapeDtypeStruct((K, 1), jnp.float32),
        ],
        scratch_shapes=[
            pltpu.VMEM((N, K), jnp.float32),
            pltpu.VMEM((N, K), jnp.float32),
            pltpu.VMEM((N, N), jnp.float8_e4m3fn),
            pltpu.VMEM((N, 1), jnp.int32),
            pltpu.VMEM((N, K), jnp.float32),
            pltpu.VMEM((N, K), jnp.float8_e4m3fn),
        ],
    )(z2, l1t, l0t, adj_matrix)

    new_assign = asg.reshape(N)

    p, ll = pl.pallas_call(
        _final_body,
        out_shape=[
            jax.ShapeDtypeStruct((K, K), jnp.float32),
            jax.ShapeDtypeStruct((1, 1), jnp.float32),
        ],
    )(m_mat, nrow, ncol)

    return resp, new_assign, p, ll.reshape(())


# speedup vs baseline: 7.5212x; 1.0502x over previous
import jax
import jax.numpy as jnp
from jax.experimental import pallas as pl
from jax.experimental.pallas import tpu as pltpu

N = 4096
K = 32
EPS = 1e-10
BR = 256
NB = N // BR
PREC = jax.lax.Precision.HIGHEST


def _em_body(z_ref, bp_ref, adj_ref,
             resp_ref, asg_ref, p_ref, ll_ref,
             l1z_scr, l0z_scr, abf_scr, zn_scr, e_scr, ebf_scr,
             m_scr, nrow_scr, ncol_scr):
    i = pl.program_id(0)

    @pl.when(i == 0)
    def _init_e():
        bp = bp_ref[...]
        l1t = jnp.log(bp + EPS).T
        l0t = jnp.log(1.0 - bp + EPS).T
        kio = jax.lax.broadcasted_iota(jnp.int32, (N, K), 1)
        onehot = (z_ref[...] == kio).astype(jnp.float32)
        l1z_scr[...] = jnp.dot(onehot, l1t,
                               preferred_element_type=jnp.float32, precision=PREC)
        l0z_scr[...] = jnp.dot(onehot, l0t,
                               preferred_element_type=jnp.float32, precision=PREC)

    @pl.when(i < NB)
    def _estep():
        a = adj_ref[...]
        abf_scr[pl.ds(i * BR, BR), :] = a.astype(jnp.float8_e4m3fn)
        logp = (jnp.dot(a, l1z_scr[...], preferred_element_type=jnp.float32)
                + jnp.dot(1.0 - a, l0z_scr[...],
                          preferred_element_type=jnp.float32))
        logp = logp - l0z_scr[pl.ds(i * BR, BR), :]
        m = jnp.max(logp, axis=1, keepdims=True)
        e = jnp.exp(logp - m)
        r = e / jnp.sum(e, axis=1, keepdims=True)
        resp_ref[...] = r
        mx = jnp.max(r, axis=1, keepdims=True)
        kio = jax.lax.broadcasted_iota(jnp.int32, (BR, K), 1)
        asg = jnp.min(jnp.where(r == mx, kio, K), axis=1, keepdims=True)
        asg_ref[...] = asg
        zn_scr[pl.ds(i * BR, BR), :] = asg

    @pl.when(i == NB)
    def _init_m():
        kio = jax.lax.broadcasted_iota(jnp.int32, (N, K), 1)
        e_full = (zn_scr[...] == kio).astype(jnp.float32)
        e_scr[...] = e_full
        ebf_scr[...] = e_full.astype(jnp.float8_e4m3fn)
        nrow_scr[...] = jnp.sum(e_full, axis=0, keepdims=True)
        ones = jnp.ones((N, 1), jnp.float32)
        ncol_scr[...] = jax.lax.dot_general(
            e_full, ones, (((0,), (0,)), ((), ())),
            preferred_element_type=jnp.float32)
        m_scr[...] = jnp.zeros((K, K), jnp.float32)

    @pl.when(i >= NB)
    def _mstep():
        j = i - NB
        ab = abf_scr[pl.ds(j * BR, BR), :]
        r = jnp.dot(ab, ebf_scr[...], preferred_element_type=jnp.float32)
        e_blk = e_scr[pl.ds(j * BR, BR), :]
        m_scr[...] += jax.lax.dot_general(
            e_blk, r, (((0,), (0,)), ((), ())),
            preferred_element_type=jnp.float32)

    @pl.when(i == 2 * NB - 1)
    def _final():
        m = m_scr[...]
        nrow = nrow_scr[...]
        ncol = ncol_scr[...]
        ri = jax.lax.broadcasted_iota(jnp.int32, (K, K), 0)
        ci = jax.lax.broadcasted_iota(jnp.int32, (K, K), 1)
        eye = ri == ci
        counts_raw = ncol * nrow - jnp.where(eye, nrow, 0.0)
        counts = jnp.where(counts_raw == 0.0, 1.0, counts_raw)
        p = m / counts
        l1p = jnp.log(p + EPS)
        l0p = jnp.log(1.0 - p + EPS)
        ll = 0.5 * jnp.sum(m * l1p + (counts_raw - m) * l0p)
        p_ref[...] = p
        ll_ref[...] = jnp.full((1, 1), ll, jnp.float32)


def kernel(adj_matrix, block_assignments, block_probs):
    z2 = block_assignments.astype(jnp.int32).reshape(N, 1)

    last = NB - 1
    resp, asg, p, ll = pl.pallas_call(
        _em_body,
        grid=(2 * NB,),
        in_specs=[
            pl.BlockSpec((N, 1), lambda i: (0, 0)),
            pl.BlockSpec((K, K), lambda i: (0, 0)),
            pl.BlockSpec((BR, N), lambda i: (jnp.minimum(i, last), 0)),
        ],
        out_specs=[
            pl.BlockSpec((BR, K), lambda i: (jnp.minimum(i, last), 0)),
            pl.BlockSpec((BR, 1), lambda i: (jnp.minimum(i, last), 0)),
            pl.BlockSpec((K, K), lambda i: (0, 0)),
            pl.BlockSpec((1, 1), lambda i: (0, 0)),
        ],
        out_shape=[
            jax.ShapeDtypeStruct((N, K), jnp.float32),
            jax.ShapeDtypeStruct((N, 1), jnp.int32),
            jax.ShapeDtypeStruct((K, K), jnp.float32),
            jax.ShapeDtypeStruct((1, 1), jnp.float32),
        ],
        scratch_shapes=[
            pltpu.VMEM((N, K), jnp.float32),
            pltpu.VMEM((N, K), jnp.float32),
            pltpu.VMEM((N, N), jnp.float8_e4m3fn),
            pltpu.VMEM((N, 1), jnp.int32),
            pltpu.VMEM((N, K), jnp.float32),
            pltpu.VMEM((N, K), jnp.float8_e4m3fn),
            pltpu.VMEM((K, K), jnp.float32),
            pltpu.VMEM((1, K), jnp.float32),
            pltpu.VMEM((K, 1), jnp.float32),
        ],
    )(z2, block_probs, adj_matrix)

    return resp, asg.reshape(N), p, ll.reshape(())


# speedup vs baseline: 8.5315x; 1.1343x over previous
import jax
import jax.numpy as jnp
from jax.experimental import pallas as pl
from jax.experimental.pallas import tpu as pltpu

N = 4096
K = 32
EPS = 1e-10
BR = 512
NB = N // BR
PREC = jax.lax.Precision.HIGHEST


def _em_body(z_ref, bp_ref, adj_ref,
             resp_ref, asg_ref, p_ref, ll_ref,
             l1z_scr, l0z_scr, abf_scr, zn_scr, e_scr, ebf_scr,
             m_scr, nrow_scr, ncol_scr):
    i = pl.program_id(0)

    @pl.when(i == 0)
    def _init_e():
        bp = bp_ref[...]
        l1t = jnp.log(bp + EPS).T
        l0t = jnp.log(1.0 - bp + EPS).T
        kio = jax.lax.broadcasted_iota(jnp.int32, (N, K), 1)
        onehot = (z_ref[...] == kio).astype(jnp.float32)
        l1z_scr[...] = jnp.dot(onehot, l1t,
                               preferred_element_type=jnp.float32, precision=PREC)
        l0z_scr[...] = jnp.dot(onehot, l0t,
                               preferred_element_type=jnp.float32, precision=PREC)

    @pl.when(i < NB)
    def _estep():
        a = adj_ref[...]
        abf_scr[pl.ds(i * BR, BR), :] = a.astype(jnp.float8_e4m3fn)
        logp = (jnp.dot(a, l1z_scr[...], preferred_element_type=jnp.float32)
                + jnp.dot(1.0 - a, l0z_scr[...],
                          preferred_element_type=jnp.float32))
        logp = logp - l0z_scr[pl.ds(i * BR, BR), :]
        m = jnp.max(logp, axis=1, keepdims=True)
        e = jnp.exp(logp - m)
        r = e / jnp.sum(e, axis=1, keepdims=True)
        resp_ref[...] = r
        mx = jnp.max(r, axis=1, keepdims=True)
        kio = jax.lax.broadcasted_iota(jnp.int32, (BR, K), 1)
        asg = jnp.min(jnp.where(r == mx, kio, K), axis=1, keepdims=True)
        asg_ref[...] = asg
        zn_scr[pl.ds(i * BR, BR), :] = asg

    @pl.when(i == NB)
    def _init_m():
        kio = jax.lax.broadcasted_iota(jnp.int32, (N, K), 1)
        e_full = (zn_scr[...] == kio).astype(jnp.float32)
        e_scr[...] = e_full
        ebf_scr[...] = e_full.astype(jnp.float8_e4m3fn)
        nrow_scr[...] = jnp.sum(e_full, axis=0, keepdims=True)
        ones = jnp.ones((N, 1), jnp.float32)
        ncol_scr[...] = jax.lax.dot_general(
            e_full, ones, (((0,), (0,)), ((), ())),
            preferred_element_type=jnp.float32)
        m_scr[...] = jnp.zeros((K, K), jnp.float32)

    @pl.when(i >= NB)
    def _mstep():
        j = i - NB
        ab = abf_scr[pl.ds(j * BR, BR), :]
        r = jnp.dot(ab, ebf_scr[...], preferred_element_type=jnp.float32)
        e_blk = e_scr[pl.ds(j * BR, BR), :]
        m_scr[...] += jax.lax.dot_general(
            e_blk, r, (((0,), (0,)), ((), ())),
            preferred_element_type=jnp.float32)

    @pl.when(i == 2 * NB - 1)
    def _final():
        m = m_scr[...]
        nrow = nrow_scr[...]
        ncol = ncol_scr[...]
        ri = jax.lax.broadcasted_iota(jnp.int32, (K, K), 0)
        ci = jax.lax.broadcasted_iota(jnp.int32, (K, K), 1)
        eye = ri == ci
        counts_raw = ncol * nrow - jnp.where(eye, nrow, 0.0)
        counts = jnp.where(counts_raw == 0.0, 1.0, counts_raw)
        p = m / counts
        l1p = jnp.log(p + EPS)
        l0p = jnp.log(1.0 - p + EPS)
        ll = 0.5 * jnp.sum(m * l1p + (counts_raw - m) * l0p)
        p_ref[...] = p
        ll_ref[...] = jnp.full((1, 1), ll, jnp.float32)


def kernel(adj_matrix, block_assignments, block_probs):
    z2 = block_assignments.astype(jnp.int32).reshape(N, 1)

    last = NB - 1
    resp, asg, p, ll = pl.pallas_call(
        _em_body,
        grid=(2 * NB,),
        in_specs=[
            pl.BlockSpec((N, 1), lambda i: (0, 0)),
            pl.BlockSpec((K, K), lambda i: (0, 0)),
            pl.BlockSpec((BR, N), lambda i: (jnp.minimum(i, last), 0)),
        ],
        out_specs=[
            pl.BlockSpec((BR, K), lambda i: (jnp.minimum(i, last), 0)),
            pl.BlockSpec((BR, 1), lambda i: (jnp.minimum(i, last), 0)),
            pl.BlockSpec((K, K), lambda i: (0, 0)),
            pl.BlockSpec((1, 1), lambda i: (0, 0)),
        ],
        out_shape=[
            jax.ShapeDtypeStruct((N, K), jnp.float32),
            jax.ShapeDtypeStruct((N, 1), jnp.int32),
            jax.ShapeDtypeStruct((K, K), jnp.float32),
            jax.ShapeDtypeStruct((1, 1), jnp.float32),
        ],
        scratch_shapes=[
            pltpu.VMEM((N, K), jnp.float32),
            pltpu.VMEM((N, K), jnp.float32),
            pltpu.VMEM((N, N), jnp.float8_e4m3fn),
            pltpu.VMEM((N, 1), jnp.int32),
            pltpu.VMEM((N, K), jnp.float32),
            pltpu.VMEM((N, K), jnp.float8_e4m3fn),
            pltpu.VMEM((K, K), jnp.float32),
            pltpu.VMEM((1, K), jnp.float32),
            pltpu.VMEM((K, 1), jnp.float32),
        ],
    )(z2, block_probs, adj_matrix)

    return resp, asg.reshape(N), p, ll.reshape(())
